# baseline (device time: 173179 ns/iter reference)
import functools

import jax
import jax.numpy as jnp
from jax import lax
from jax.experimental import pallas as pl
from jax.experimental.pallas import tpu as pltpu

N_DEV = 8
B_, S_, D_, N_ = 8, 512, 512, 16


def kernel(x, A, B, C):
    def body(x_ref, A_ref, B_ref, C_ref, out_ref,
             f_ref, carry_ref, send_ref, send_sem, recv_sem):
        my = lax.axis_index("i")
        left = (my - 1) % N_DEV
        right = (my + 1) % N_DEV

        barrier_sem = pltpu.get_barrier_semaphore()
        for nbr in (left, right):
            pl.semaphore_signal(
                barrier_sem, inc=1,
                device_id=(nbr,), device_id_type=pl.DeviceIdType.MESH,
            )
        pl.semaphore_wait(barrier_sem, 2)

        dA_T = jnp.exp(A_ref[:, :]).T

        def step(t, h):
            xt = x_ref[:, pl.ds(t, 1), :][:, 0, :]
            Bt = B_ref[:, pl.ds(t, 1), :][:, 0, :]
            Ct = C_ref[:, pl.ds(t, 1), :][:, 0, :]
            h = h * dA_T[None, :, :] + xt[:, None, :] * Bt[:, :, None]
            yt = jnp.sum(h * Ct[:, :, None], axis=1)
            out_ref[:, pl.ds(t, 1), :] = yt[:, None, :]
            return h

        h0 = jnp.zeros((B_, N_, D_), jnp.float32)
        f_ref[...] = lax.fori_loop(0, S_, step, h0)

        @pl.when(my == 0)
        def _():
            carry_ref[...] = jnp.zeros((B_, N_, D_), jnp.float32)

        @pl.when(my > 0)
        def _():
            recv = pltpu.make_async_remote_copy(
                src_ref=send_ref, dst_ref=carry_ref,
                send_sem=send_sem, recv_sem=recv_sem,
                device_id=(left,), device_id_type=pl.DeviceIdType.MESH,
            )
            recv.wait_recv()

        @pl.when(my < N_DEV - 1)
        def _():
            dApowS = jnp.exp(A_ref[:, :].T * float(S_))
            send_ref[...] = carry_ref[...] * dApowS[None, :, :] + f_ref[...]
            send = pltpu.make_async_remote_copy(
                src_ref=send_ref, dst_ref=carry_ref,
                send_sem=send_sem, recv_sem=recv_sem,
                device_id=(right,), device_id_type=pl.DeviceIdType.MESH,
            )
            send.start()
            send.wait_send()

        def fix_step(t, m):
            m = m * dA_T[None, :, :]
            Ct = C_ref[:, pl.ds(t, 1), :][:, 0, :]
            yfix = jnp.sum(m * Ct[:, :, None], axis=1)
            cur = out_ref[:, pl.ds(t, 1), :]
            out_ref[:, pl.ds(t, 1), :] = cur + yfix[:, None, :]
            return m

        lax.fori_loop(0, S_, fix_step, carry_ref[...])

    return pl.pallas_call(
        body,
        out_shape=jax.ShapeDtypeStruct((B_, S_, D_), jnp.float32),
        in_specs=[
            pl.BlockSpec(memory_space=pltpu.VMEM),
            pl.BlockSpec(memory_space=pltpu.VMEM),
            pl.BlockSpec(memory_space=pltpu.VMEM),
            pl.BlockSpec(memory_space=pltpu.VMEM),
        ],
        out_specs=pl.BlockSpec(memory_space=pltpu.VMEM),
        scratch_shapes=[
            pltpu.VMEM((B_, N_, D_), jnp.float32),
            pltpu.VMEM((B_, N_, D_), jnp.float32),
            pltpu.VMEM((B_, N_, D_), jnp.float32),
            pltpu.SemaphoreType.DMA,
            pltpu.SemaphoreType.DMA,
        ],
        compiler_params=pltpu.CompilerParams(collective_id=0),
    )(x, A, B, C)


# device time: 124691 ns/iter; 1.3889x vs baseline; 1.3889x over previous
import jax
import jax.numpy as jnp
from jax import lax
from jax.experimental import pallas as pl
from jax.experimental.pallas import tpu as pltpu

N_DEV = 8
B_, S_, D_, N_ = 8, 512, 512, 16
NC, L = 16, 32


def kernel(x, A, B, C):
    def body(x_ref, A_ref, B_ref, C_ref, out_ref,
             xr_ref, br_ref, cr_ref, yr_ref,
             H_ref, P_ref, f_ref, carry_ref, send_ref, send_sem, recv_sem):
        my = lax.axis_index("i")
        left = (my - 1) % N_DEV
        right = (my + 1) % N_DEV

        barrier_sem = pltpu.get_barrier_semaphore()
        for nbr in (left, right):
            pl.semaphore_signal(
                barrier_sem, inc=1,
                device_id=(nbr,), device_id_type=pl.DeviceIdType.MESH,
            )
        pl.semaphore_wait(barrier_sem, 2)

        dA_T = jnp.exp(A_ref[:, :]).T
        dAL = jnp.exp(A_ref[:, :].T * float(L))

        for c in range(NC):
            sl = slice(c * L, (c + 1) * L)
            xr_ref[:, :, c, :] = x_ref[:, sl, :]
            br_ref[:, :, c, :] = B_ref[:, sl, :]
            cr_ref[:, :, c, :] = C_ref[:, sl, :]

        H_ref[...] = jnp.zeros((B_, NC, N_, D_), jnp.float32)

        def stepA(t, _):
            xt = xr_ref[:, pl.ds(t, 1), :, :][:, 0]
            Bt = br_ref[:, pl.ds(t, 1), :, :][:, 0]
            Ct = cr_ref[:, pl.ds(t, 1), :, :][:, 0]
            H = (H_ref[...] * dA_T[None, None, :, :]
                 + xt[:, :, None, :] * Bt[:, :, :, None])
            H_ref[...] = H
            yt = jnp.sum(H * Ct[:, :, :, None], axis=2)
            yr_ref[:, pl.ds(t, 1), :, :] = yt[:, None]
            return 0

        lax.fori_loop(0, L, stepA, 0)

        def stepB(c, p):
            P_ref[:, pl.ds(c, 1), :, :] = p[:, None, :, :]
            Fc = H_ref[:, pl.ds(c, 1), :, :][:, 0, :, :]
            return p * dAL[None, :, :] + Fc

        f_dev = lax.fori_loop(0, NC, stepB, jnp.zeros((B_, N_, D_), jnp.float32))
        f_ref[...] = f_dev

        @pl.when(my == 0)
        def _():
            carry_ref[...] = jnp.zeros((B_, N_, D_), jnp.float32)

        @pl.when(my > 0)
        def _():
            recv = pltpu.make_async_remote_copy(
                src_ref=send_ref, dst_ref=carry_ref,
                send_sem=send_sem, recv_sem=recv_sem,
                device_id=(left,), device_id_type=pl.DeviceIdType.MESH,
            )
            recv.wait_recv()

        @pl.when(my < N_DEV - 1)
        def _():
            dApowS = jnp.exp(A_ref[:, :].T * float(S_))
            send_ref[...] = carry_ref[...] * dApowS[None, :, :] + f_ref[...]
            send = pltpu.make_async_remote_copy(
                src_ref=send_ref, dst_ref=carry_ref,
                send_sem=send_sem, recv_sem=recv_sem,
                device_id=(right,), device_id_type=pl.DeviceIdType.MESH,
            )
            send.start()
            send.wait_send()

        cL = (lax.broadcasted_iota(jnp.int32, (NC, N_, D_), 0)
              .astype(jnp.float32) * float(L))
        dApow_cL = jnp.exp(A_ref[:, :].T[None, :, :] * cL)
        H_ref[...] = (P_ref[...]
                      + dApow_cL[None, :, :, :] * carry_ref[...][:, None, :, :])

        def stepC(t, _):
            M = H_ref[...] * dA_T[None, None, :, :]
            H_ref[...] = M
            Ct = cr_ref[:, pl.ds(t, 1), :, :][:, 0]
            yfix = jnp.sum(M * Ct[:, :, :, None], axis=2)
            cur = yr_ref[:, pl.ds(t, 1), :, :]
            yr_ref[:, pl.ds(t, 1), :, :] = cur + yfix[:, None]
            return 0

        lax.fori_loop(0, L, stepC, 0)

        for c in range(NC):
            out_ref[:, slice(c * L, (c + 1) * L), :] = yr_ref[:, :, c, :]

    return pl.pallas_call(
        body,
        out_shape=jax.ShapeDtypeStruct((B_, S_, D_), jnp.float32),
        in_specs=[
            pl.BlockSpec(memory_space=pltpu.VMEM),
            pl.BlockSpec(memory_space=pltpu.VMEM),
            pl.BlockSpec(memory_space=pltpu.VMEM),
            pl.BlockSpec(memory_space=pltpu.VMEM),
        ],
        out_specs=pl.BlockSpec(memory_space=pltpu.VMEM),
        scratch_shapes=[
            pltpu.VMEM((B_, L, NC, D_), jnp.float32),
            pltpu.VMEM((B_, L, NC, N_), jnp.float32),
            pltpu.VMEM((B_, L, NC, N_), jnp.float32),
            pltpu.VMEM((B_, L, NC, D_), jnp.float32),
            pltpu.VMEM((B_, NC, N_, D_), jnp.float32),
            pltpu.VMEM((B_, NC, N_, D_), jnp.float32),
            pltpu.VMEM((B_, N_, D_), jnp.float32),
            pltpu.VMEM((B_, N_, D_), jnp.float32),
            pltpu.VMEM((B_, N_, D_), jnp.float32),
            pltpu.SemaphoreType.DMA,
            pltpu.SemaphoreType.DMA,
        ],
        compiler_params=pltpu.CompilerParams(
            collective_id=0, vmem_limit_bytes=100 * 1024 * 1024,
        ),
    )(x, A, B, C)


# device time: 24561 ns/iter; 7.0510x vs baseline; 5.0768x over previous
import os

import jax
import jax.numpy as jnp
from jax import lax
from jax.experimental import pallas as pl
from jax.experimental.pallas import tpu as pltpu

N_DEV = 8
B_, S_, D_, N_ = 8, 512, 512, 16
NC, L = 16, 32

_ABL = int(os.environ.get("ABL", "0"))
ABLATE_B_CHAIN_C = _ABL >= 1
ABLATE_A = _ABL >= 2


def kernel(x, A, B, C):
    def body(x_ref, A_ref, B_ref, C_ref, out_ref,
             xr_ref, br_ref, cr_ref, yr_ref,
             H_ref, P_ref, f_ref, carry_ref, send_ref, send_sem, recv_sem):
        my = lax.axis_index("i")
        left = (my - 1) % N_DEV
        right = (my + 1) % N_DEV

        barrier_sem = pltpu.get_barrier_semaphore()
        for nbr in (left, right):
            pl.semaphore_signal(
                barrier_sem, inc=1,
                device_id=(nbr,), device_id_type=pl.DeviceIdType.MESH,
            )
        pl.semaphore_wait(barrier_sem, 2)

        dA_T = jnp.exp(A_ref[:, :]).T
        dAL = jnp.exp(A_ref[:, :].T * float(L))

        for c in range(NC):
            sl = slice(c * L, (c + 1) * L)
            xr_ref[:, :, c, :] = x_ref[:, sl, :]
            br_ref[:, :, c, :] = B_ref[:, sl, :]
            cr_ref[:, :, c, :] = C_ref[:, sl, :]

        H_ref[...] = jnp.zeros((B_, NC, N_, D_), jnp.float32)

        def stepA(t, _):
            xt = xr_ref[:, pl.ds(t, 1), :, :][:, 0]
            Bt = br_ref[:, pl.ds(t, 1), :, :][:, 0]
            Ct = cr_ref[:, pl.ds(t, 1), :, :][:, 0]
            H = (H_ref[...] * dA_T[None, None, :, :]
                 + xt[:, :, None, :] * Bt[:, :, :, None])
            H_ref[...] = H
            yt = jnp.sum(H * Ct[:, :, :, None], axis=2)
            yr_ref[:, pl.ds(t, 1), :, :] = yt[:, None]
            return 0

        if not ABLATE_A:
            lax.fori_loop(0, L, stepA, 0)

        if not ABLATE_B_CHAIN_C:
            def stepB(c, p):
                P_ref[:, pl.ds(c, 1), :, :] = p[:, None, :, :]
                Fc = H_ref[:, pl.ds(c, 1), :, :][:, 0, :, :]
                return p * dAL[None, :, :] + Fc

            f_dev = lax.fori_loop(
                0, NC, stepB, jnp.zeros((B_, N_, D_), jnp.float32))
            f_ref[...] = f_dev

            @pl.when(my == 0)
            def _():
                carry_ref[...] = jnp.zeros((B_, N_, D_), jnp.float32)

            @pl.when(my > 0)
            def _():
                recv = pltpu.make_async_remote_copy(
                    src_ref=send_ref, dst_ref=carry_ref,
                    send_sem=send_sem, recv_sem=recv_sem,
                    device_id=(left,), device_id_type=pl.DeviceIdType.MESH,
                )
                recv.wait_recv()

            @pl.when(my < N_DEV - 1)
            def _():
                dApowS = jnp.exp(A_ref[:, :].T * float(S_))
                send_ref[...] = (carry_ref[...] * dApowS[None, :, :]
                                 + f_ref[...])
                send = pltpu.make_async_remote_copy(
                    src_ref=send_ref, dst_ref=carry_ref,
                    send_sem=send_sem, recv_sem=recv_sem,
                    device_id=(right,), device_id_type=pl.DeviceIdType.MESH,
                )
                send.start()
                send.wait_send()

            cL = (lax.broadcasted_iota(jnp.int32, (NC, N_, D_), 0)
                  .astype(jnp.float32) * float(L))
            dApow_cL = jnp.exp(A_ref[:, :].T[None, :, :] * cL)
            H_ref[...] = (P_ref[...]
                          + dApow_cL[None, :, :, :]
                          * carry_ref[...][:, None, :, :])

            def stepC(t, _):
                M = H_ref[...] * dA_T[None, None, :, :]
                H_ref[...] = M
                Ct = cr_ref[:, pl.ds(t, 1), :, :][:, 0]
                yfix = jnp.sum(M * Ct[:, :, :, None], axis=2)
                cur = yr_ref[:, pl.ds(t, 1), :, :]
                yr_ref[:, pl.ds(t, 1), :, :] = cur + yfix[:, None]
                return 0

            lax.fori_loop(0, L, stepC, 0)

        for c in range(NC):
            out_ref[:, slice(c * L, (c + 1) * L), :] = yr_ref[:, :, c, :]

    return pl.pallas_call(
        body,
        out_shape=jax.ShapeDtypeStruct((B_, S_, D_), jnp.float32),
        in_specs=[
            pl.BlockSpec(memory_space=pltpu.VMEM),
            pl.BlockSpec(memory_space=pltpu.VMEM),
            pl.BlockSpec(memory_space=pltpu.VMEM),
            pl.BlockSpec(memory_space=pltpu.VMEM),
        ],
        out_specs=pl.BlockSpec(memory_space=pltpu.VMEM),
        scratch_shapes=[
            pltpu.VMEM((B_, L, NC, D_), jnp.float32),
            pltpu.VMEM((B_, L, NC, N_), jnp.float32),
            pltpu.VMEM((B_, L, NC, N_), jnp.float32),
            pltpu.VMEM((B_, L, NC, D_), jnp.float32),
            pltpu.VMEM((B_, NC, N_, D_), jnp.float32),
            pltpu.VMEM((B_, NC, N_, D_), jnp.float32),
            pltpu.VMEM((B_, N_, D_), jnp.float32),
            pltpu.VMEM((B_, N_, D_), jnp.float32),
            pltpu.VMEM((B_, N_, D_), jnp.float32),
            pltpu.SemaphoreType.DMA,
            pltpu.SemaphoreType.DMA,
        ],
        compiler_params=pltpu.CompilerParams(
            collective_id=0, vmem_limit_bytes=100 * 1024 * 1024,
        ),
    )(x, A, B, C)
